# baseline (speedup 1.0000x reference)
import functools

import jax
import jax.numpy as jnp
from jax import lax
from jax.experimental import pallas as pl
from jax.experimental.pallas import tpu as pltpu
from jax.experimental.pallas import tpu_sc as plsc

N_TPL = 342
D = 128

NC = 2
NS = 16
NP = 2

S = 1024
RB = 2048
C = 128
TRASH = 128
NBUF = 3
SPAN = 448


def _mm_body(x_ref, wrelT_ref, wrootT_ref, b_ref, y_ref, z_ref):
    y_ref[...] = jnp.dot(x_ref[...], wrelT_ref[...],
                         preferred_element_type=jnp.float32,
                         precision=lax.Precision.HIGHEST)
    z_ref[...] = jnp.dot(x_ref[...], wrootT_ref[...],
                         preferred_element_type=jnp.float32,
                         precision=lax.Precision.HIGHEST) + b_ref[...]


def _tc_matmuls(x, W_rel, b_rel, W_root):
    n = x.shape[0]
    BM = 2048
    return pl.pallas_call(
        _mm_body,
        grid=(pl.cdiv(n, BM),),
        in_specs=[
            pl.BlockSpec((BM, D), lambda i: (i, 0)),
            pl.BlockSpec((D, D), lambda i: (0, 0)),
            pl.BlockSpec((D, D), lambda i: (0, 0)),
            pl.BlockSpec((1, D), lambda i: (0, 0)),
        ],
        out_specs=[
            pl.BlockSpec((BM, D), lambda i: (i, 0)),
            pl.BlockSpec((BM, D), lambda i: (i, 0)),
        ],
        out_shape=[
            jax.ShapeDtypeStruct((n, D), jnp.float32),
            jax.ShapeDtypeStruct((n, D), jnp.float32),
        ],
    )(x, W_rel.T, W_root.T, b_rel[None, :])


def _make_sc_scatter(n, ept):
    rng = n // (NC * NP)
    nseg = ept // S
    acc_rows = rng + TRASH
    SEGW = 3 * S
    mesh = plsc.VectorSubcoreMesh(core_axis_name="c", subcore_axis_name="s")

    @functools.partial(
        pl.kernel,
        mesh=mesh,
        out_type=jax.ShapeDtypeStruct((n, D), jnp.float32),
        compiler_params=pltpu.CompilerParams(needs_layout_passes=False),
        scratch_types=[
            pltpu.VMEM((2 * SEGW,), jnp.int32),
            pltpu.VMEM((RB,), jnp.int32),
            pltpu.VMEM((RB,), jnp.int32),
            pltpu.VMEM((RB,), jnp.float32),
            pltpu.VMEM((NBUF, C), jnp.int32),
            pltpu.VMEM((NBUF * C, D), jnp.float32),
            pltpu.VMEM_SHARED((acc_rows, D), jnp.float32),
            pltpu.SemaphoreType.DMA((2,)),
            pltpu.SemaphoreType.DMA((NBUF,)),
            pltpu.SemaphoreType.DMA((NBUF,)),
        ],
    )
    def sc_kernel(y_hbm, z_hbm, packed_hbm, out_hbm,
                  seg_v, sring, dring, wring, didx_st,
                  rows_v, acc, psem, gsem, ssem):
        c = lax.axis_index("c")
        s = lax.axis_index("s")
        iota = lax.iota(jnp.int32, 16)

        def issue_seg(g):
            b = g & 1
            pltpu.async_copy(
                packed_hbm.at[pl.ds(
                    pl.multiple_of((s * nseg + g) * SEGW, 8), SEGW)],
                seg_v.at[pl.ds(pl.multiple_of(b * SEGW, 8), SEGW)],
                psem.at[b])

        def wait_seg(g):
            b = g & 1
            pltpu.make_async_copy(
                packed_hbm.at[pl.ds(0, SEGW)],
                seg_v.at[pl.ds(pl.multiple_of(b * SEGW, 8), SEGW)],
                psem.at[b]).wait()

        def issue_gather(q):
            p = lax.rem(q, NBUF)
            cb = pl.multiple_of((q * C) & (RB - 1), 8)
            rb = pl.multiple_of(p * C, 8)
            pltpu.async_copy(y_hbm.at[sring.at[pl.ds(cb, C)]],
                             rows_v.at[pl.ds(rb, C)], gsem.at[p])

        def finish_chunk(q):
            p = lax.rem(q, NBUF)
            cb = pl.multiple_of((q * C) & (RB - 1), 8)
            rb = pl.multiple_of(p * C, 8)
            pltpu.make_async_copy(y_hbm.at[pl.ds(0, C)],
                                  rows_v.at[pl.ds(rb, C)], gsem.at[p]).wait()
            for k in range(C // 16):
                didx_st[p, pl.ds(k * 16, 16)] = dring[pl.ds(cb + k * 16, 16)]

            @plsc.parallel_loop(0, C, unroll=4)
            def scale(e):
                wsp = plsc.load_gather(wring, [jnp.broadcast_to(cb + e, (16,))])
                rref = rows_v.at[rb + e]
                for j in range(D // 16):
                    rref[pl.ds(j * 16, 16)] = rref[pl.ds(j * 16, 16)] * wsp

            pltpu.async_copy(rows_v.at[pl.ds(rb, C)], acc.at[didx_st.at[p]],
                             ssem.at[p], add=True)

        def wait_scatter(q):
            p = lax.rem(q, NBUF)
            rb = pl.multiple_of(p * C, 8)
            pltpu.make_async_copy(rows_v.at[pl.ds(rb, C)],
                                  acc.at[didx_st.at[p]], ssem.at[p]).wait()

        def fin_loop(lo, hi):
            def fin(q, cc):
                finish_chunk(q)
                return cc
            lax.fori_loop(lo, hi, fin, 0)

        for p_ in range(NP):
            qb = (c * NP + p_) * rng

            issue_seg(0)
            zs = pl.multiple_of(jnp.minimum(s * SPAN, rng - SPAN), 8)
            pltpu.sync_copy(z_hbm.at[pl.ds(pl.multiple_of(qb + zs, 8), SPAN)],
                            acc.at[pl.ds(zs, SPAN)])
            plsc.subcore_barrier()

            def segment(g, carry):
                off, issued, finished = carry
                b = g & 1
                sb = pl.multiple_of(b * SEGW, 8)
                wait_seg(g)

                @pl.when(g + 1 < nseg)
                def _():
                    issue_seg(g + 1)

                for v in range(S // 16):
                    srcv = seg_v[pl.ds(sb + v * 16, 16)]
                    dstv = seg_v[pl.ds(sb + S + v * 16, 16)]
                    ewv = plsc.bitcast(seg_v[pl.ds(sb + 2 * S + v * 16, 16)],
                                       jnp.float32)
                    dl = dstv - qb
                    ok = (dl >= 0) & (dl < rng)
                    cum = plsc.cumsum(ok.astype(jnp.int32))
                    pos = (off + cum - 1) & (RB - 1)
                    plsc.store_scatter(sring, [pos], srcv, mask=ok)
                    plsc.store_scatter(dring, [pos], dl, mask=ok)
                    plsc.store_scatter(wring, [pos], ewv, mask=ok)
                    off = off + plsc.all_reduce_population_count(ok)

                fin_loop(finished, issued)
                navail = off[0] // C
                lim = jnp.minimum(navail, issued + NBUF)

                def iss(q, cc):
                    @pl.when(q >= NBUF)
                    def _():
                        wait_scatter(q - NBUF)
                    issue_gather(q)
                    return cc

                lax.fori_loop(issued, lim, iss, 0)

                def over(q, cc):
                    finish_chunk(q - NBUF)
                    wait_scatter(q - NBUF)
                    issue_gather(q)
                    return cc

                lax.fori_loop(lim, navail, over, 0)
                finished2 = issued + jnp.maximum(navail - lim, 0)
                return off, navail, finished2

            off, issued, finished = lax.fori_loop(
                0, nseg, segment,
                (jnp.zeros((16,), jnp.int32), jnp.int32(0), jnp.int32(0)))

            wid = s * NC + c
            for v in range(C // 16):
                pv = (off + (v * 16) + iota) & (RB - 1)
                plsc.store_scatter(sring, [pv], (pv * 61 + wid * 997) % n)
                plsc.store_scatter(dring, [pv], rng + (pv & (TRASH - 1)))
                plsc.store_scatter(wring, [pv], jnp.zeros((16,), jnp.float32))
            total = (off[0] + C - 1) // C
            fin_loop(finished, issued)

            def tail(q, cc):
                @pl.when(q >= NBUF)
                def _():
                    wait_scatter(q - NBUF)
                issue_gather(q)
                finish_chunk(q)
                return cc

            lax.fori_loop(issued, total, tail, 0)

            def dw(q, cc):
                wait_scatter(q)
                return cc

            lax.fori_loop(jnp.maximum(total - NBUF, 0), total, dw, 0)
            plsc.subcore_barrier()

            os_ = pl.multiple_of(jnp.minimum(s * SPAN, rng - SPAN), 8)
            for q in range(SPAN // 112):
                pltpu.sync_copy(acc.at[pl.ds(os_ + q * 112, 112)],
                                rows_v.at[pl.ds(0, 112)])
                pltpu.sync_copy(
                    rows_v.at[pl.ds(0, 112)],
                    out_hbm.at[pl.ds(
                        pl.multiple_of(qb + os_ + q * 112, 8), 112)])
            plsc.subcore_barrier()

    return sc_kernel


def kernel(x, edge_index, edge_weights, W_rel, b_rel, W_root):
    n = x.shape[0]
    npad = (n + 31) // 32 * 32
    e_total = edge_index.shape[1]
    ept = ((e_total + NS - 1) // NS + S - 1) // S * S
    epad = ept * NS

    src = edge_index[0]
    dst = edge_index[1]
    pad = epad - e_total
    ew_full = jnp.tile(edge_weights, (e_total + N_TPL - 1) // N_TPL)[:e_total]
    srcp = jnp.concatenate([src, jnp.arange(pad, dtype=jnp.int32) % n])
    dstp = jnp.concatenate([dst, jnp.full((pad,), npad, jnp.int32)])
    ewp = jnp.concatenate([ew_full, jnp.zeros((pad,), jnp.float32)])
    ewbits = jax.lax.bitcast_convert_type(ewp, jnp.int32)
    packed = jnp.stack([srcp.reshape(-1, S), dstp.reshape(-1, S),
                        ewbits.reshape(-1, S)], axis=1).reshape(-1)

    if npad != n:
        x = jnp.pad(x, ((0, npad - n), (0, 0)))
    y, z = _tc_matmuls(x, W_rel, b_rel, W_root)
    out = _make_sc_scatter(npad, ept)(y, z, packed)
    return out[:n]


# speedup vs baseline: 1.0342x; 1.0342x over previous
import functools

import jax
import jax.numpy as jnp
from jax import lax
from jax.experimental import pallas as pl
from jax.experimental.pallas import tpu as pltpu
from jax.experimental.pallas import tpu_sc as plsc

N_TPL = 342
D = 128

NC = 2
NS = 16
NP = 2

S = 1024
RB = 2048
C = 128
TRASH = 128
NBUF = 3
SPAN = 448


def _mm_body(x_ref, wrelT_ref, wrootT_ref, b_ref, y_ref, z_ref):
    y_ref[...] = jnp.dot(x_ref[...], wrelT_ref[...],
                         preferred_element_type=jnp.float32,
                         precision=lax.Precision.HIGHEST)
    z_ref[...] = jnp.dot(x_ref[...], wrootT_ref[...],
                         preferred_element_type=jnp.float32,
                         precision=lax.Precision.HIGHEST) + b_ref[...]


def _tc_matmuls(x, W_rel, b_rel, W_root, n_out):
    BM = 2048
    return pl.pallas_call(
        _mm_body,
        grid=(pl.cdiv(n_out, BM),),
        in_specs=[
            pl.BlockSpec((BM, D), lambda i: (i, 0)),
            pl.BlockSpec((D, D), lambda i: (0, 0)),
            pl.BlockSpec((D, D), lambda i: (0, 0)),
            pl.BlockSpec((1, D), lambda i: (0, 0)),
        ],
        out_specs=[
            pl.BlockSpec((BM, D), lambda i: (i, 0)),
            pl.BlockSpec((BM, D), lambda i: (i, 0)),
        ],
        out_shape=[
            jax.ShapeDtypeStruct((n_out, D), jnp.float32),
            jax.ShapeDtypeStruct((n_out, D), jnp.float32),
        ],
    )(x, W_rel.T, W_root.T, b_rel[None, :])


def _make_sc_scatter(n, ept):
    rng = n // (NC * NP)
    nseg = ept // S
    acc_rows = rng + TRASH
    SEGW = 3 * S
    mesh = plsc.VectorSubcoreMesh(core_axis_name="c", subcore_axis_name="s")

    @functools.partial(
        pl.kernel,
        mesh=mesh,
        out_type=jax.ShapeDtypeStruct((n, D), jnp.float32),
        compiler_params=pltpu.CompilerParams(needs_layout_passes=False),
        scratch_types=[
            pltpu.VMEM((2 * SEGW,), jnp.int32),
            pltpu.VMEM((RB,), jnp.int32),
            pltpu.VMEM((RB,), jnp.int32),
            pltpu.VMEM((RB,), jnp.float32),
            pltpu.VMEM((NBUF, C), jnp.int32),
            pltpu.VMEM((NBUF * C, D), jnp.float32),
            pltpu.VMEM_SHARED((acc_rows, D), jnp.float32),
            pltpu.SemaphoreType.DMA((2,)),
            pltpu.SemaphoreType.DMA((NBUF,)),
            pltpu.SemaphoreType.DMA((NBUF,)),
        ],
    )
    def sc_kernel(y_hbm, z_hbm, packed_hbm, out_hbm,
                  seg_v, sring, dring, wring, didx_st,
                  rows_v, acc, psem, gsem, ssem):
        c = lax.axis_index("c")
        s = lax.axis_index("s")
        iota = lax.iota(jnp.int32, 16)

        def issue_seg(g):
            b = g & 1
            pltpu.async_copy(
                packed_hbm.at[pl.ds(
                    pl.multiple_of((s * nseg + g) * SEGW, 8), SEGW)],
                seg_v.at[pl.ds(pl.multiple_of(b * SEGW, 8), SEGW)],
                psem.at[b])

        def wait_seg(g):
            b = g & 1
            pltpu.make_async_copy(
                packed_hbm.at[pl.ds(0, SEGW)],
                seg_v.at[pl.ds(pl.multiple_of(b * SEGW, 8), SEGW)],
                psem.at[b]).wait()

        def issue_gather(q):
            p = lax.rem(q, NBUF)
            cb = pl.multiple_of((q * C) & (RB - 1), 8)
            rb = pl.multiple_of(p * C, 8)
            pltpu.async_copy(y_hbm.at[sring.at[pl.ds(cb, C)]],
                             rows_v.at[pl.ds(rb, C)], gsem.at[p])

        def finish_chunk(q):
            p = lax.rem(q, NBUF)
            cb = pl.multiple_of((q * C) & (RB - 1), 8)
            rb = pl.multiple_of(p * C, 8)
            pltpu.make_async_copy(y_hbm.at[pl.ds(0, C)],
                                  rows_v.at[pl.ds(rb, C)], gsem.at[p]).wait()
            for k in range(C // 16):
                didx_st[p, pl.ds(k * 16, 16)] = dring[pl.ds(cb + k * 16, 16)]

            @plsc.parallel_loop(0, C, unroll=4)
            def scale(e):
                wsp = plsc.load_gather(wring, [jnp.broadcast_to(cb + e, (16,))])
                rref = rows_v.at[rb + e]
                for j in range(D // 16):
                    rref[pl.ds(j * 16, 16)] = rref[pl.ds(j * 16, 16)] * wsp

            pltpu.async_copy(rows_v.at[pl.ds(rb, C)], acc.at[didx_st.at[p]],
                             ssem.at[p], add=True)

        def wait_scatter(q):
            p = lax.rem(q, NBUF)
            rb = pl.multiple_of(p * C, 8)
            pltpu.make_async_copy(rows_v.at[pl.ds(rb, C)],
                                  acc.at[didx_st.at[p]], ssem.at[p]).wait()

        def fin_loop(lo, hi):
            def fin(q, cc):
                finish_chunk(q)
                return cc
            lax.fori_loop(lo, hi, fin, 0)

        for p_ in range(NP):
            qb = (c * NP + p_) * rng

            issue_seg(0)
            zs = pl.multiple_of(jnp.minimum(s * SPAN, rng - SPAN), 8)
            pltpu.sync_copy(z_hbm.at[pl.ds(pl.multiple_of(qb + zs, 8), SPAN)],
                            acc.at[pl.ds(zs, SPAN)])
            plsc.subcore_barrier()

            def segment(g, carry):
                off, issued, finished = carry
                b = g & 1
                sb = pl.multiple_of(b * SEGW, 8)
                wait_seg(g)

                @pl.when(g + 1 < nseg)
                def _():
                    issue_seg(g + 1)

                for v in range(S // 16):
                    srcv = seg_v[pl.ds(sb + v * 16, 16)]
                    dstv = seg_v[pl.ds(sb + S + v * 16, 16)]
                    ewv = plsc.bitcast(seg_v[pl.ds(sb + 2 * S + v * 16, 16)],
                                       jnp.float32)
                    dl = dstv - qb
                    ok = (dl >= 0) & (dl < rng)
                    cum = plsc.cumsum(ok.astype(jnp.int32))
                    pos = (off + cum - 1) & (RB - 1)
                    plsc.store_scatter(sring, [pos], srcv, mask=ok)
                    plsc.store_scatter(dring, [pos], dl, mask=ok)
                    plsc.store_scatter(wring, [pos], ewv, mask=ok)
                    off = off + plsc.all_reduce_population_count(ok)

                fin_loop(finished, issued)
                navail = off[0] // C
                lim = jnp.minimum(navail, issued + NBUF)

                def iss(q, cc):
                    @pl.when(q >= NBUF)
                    def _():
                        wait_scatter(q - NBUF)
                    issue_gather(q)
                    return cc

                lax.fori_loop(issued, lim, iss, 0)

                def over(q, cc):
                    finish_chunk(q - NBUF)
                    wait_scatter(q - NBUF)
                    issue_gather(q)
                    return cc

                lax.fori_loop(lim, navail, over, 0)
                finished2 = issued + jnp.maximum(navail - lim, 0)
                return off, navail, finished2

            off, issued, finished = lax.fori_loop(
                0, nseg, segment,
                (jnp.zeros((16,), jnp.int32), jnp.int32(0), jnp.int32(0)))

            wid = s * NC + c
            for v in range(C // 16):
                pv = (off + (v * 16) + iota) & (RB - 1)
                plsc.store_scatter(sring, [pv], (pv * 61 + wid * 997) % n)
                plsc.store_scatter(dring, [pv], rng + (pv & (TRASH - 1)))
                plsc.store_scatter(wring, [pv], jnp.zeros((16,), jnp.float32))
            total = (off[0] + C - 1) // C
            fin_loop(finished, issued)

            def tail(q, cc):
                @pl.when(q >= NBUF)
                def _():
                    wait_scatter(q - NBUF)
                issue_gather(q)
                finish_chunk(q)
                return cc

            lax.fori_loop(issued, total, tail, 0)

            def dw(q, cc):
                wait_scatter(q)
                return cc

            lax.fori_loop(jnp.maximum(total - NBUF, 0), total, dw, 0)
            plsc.subcore_barrier()

            os_ = pl.multiple_of(jnp.minimum(s * SPAN, rng - SPAN), 8)
            pltpu.sync_copy(acc.at[pl.ds(os_, SPAN)],
                            out_hbm.at[pl.ds(pl.multiple_of(qb + os_, 8),
                                             SPAN)])
            plsc.subcore_barrier()

    return sc_kernel


def kernel(x, edge_index, edge_weights, W_rel, b_rel, W_root):
    n = x.shape[0]
    npad = (n + 31) // 32 * 32
    e_total = edge_index.shape[1]
    ept = ((e_total + NS - 1) // NS + S - 1) // S * S
    epad = ept * NS

    src = edge_index[0]
    dst = edge_index[1]
    pad = epad - e_total
    ew_full = jnp.tile(edge_weights, (e_total + N_TPL - 1) // N_TPL)[:e_total]
    srcp = jnp.concatenate([src, jnp.arange(pad, dtype=jnp.int32) % n])
    dstp = jnp.concatenate([dst, jnp.full((pad,), npad, jnp.int32)])
    ewp = jnp.concatenate([ew_full, jnp.zeros((pad,), jnp.float32)])
    ewbits = jax.lax.bitcast_convert_type(ewp, jnp.int32)
    packed = jnp.stack([srcp.reshape(-1, S), dstp.reshape(-1, S),
                        ewbits.reshape(-1, S)], axis=1).reshape(-1)

    y, z = _tc_matmuls(x, W_rel, b_rel, W_root, npad)
    out = _make_sc_scatter(npad, ept)(y, z, packed)
    return out[:n]
